# baseline (speedup 1.0000x reference)
import functools

import jax
import jax.numpy as jnp
from jax import lax
from jax.experimental import pallas as pl
from jax.experimental.pallas import tpu as pltpu
from jax.experimental.pallas import tpu_sc as plsc

_NUM_Q = 100000
_D = 200
_B = 4096
_L = 50
_LP = 56
_DB = _D - 128

_info = plsc.get_sparse_core_info()
_NC = _info.num_cores
_NS = _info.num_subcores
_NW = _NC * _NS
_ROWS_W = _B // _NW
_NBUF = 4
_GROUPS = _ROWS_W // _NBUF

_mesh = plsc.VectorSubcoreMesh(core_axis_name="c", subcore_axis_name="s")


@functools.partial(
    pl.kernel,
    out_type=(
        jax.ShapeDtypeStruct((_B, _LP, 128), jnp.float32),
        jax.ShapeDtypeStruct((_B, _LP, 128), jnp.float32),
    ),
    mesh=_mesh,
    scratch_types=[
        pltpu.VMEM((1, _ROWS_W, _LP), jnp.int32),
        pltpu.VMEM((_LP, 128), jnp.float32),
        pltpu.VMEM((_LP, 128), jnp.float32),
        pltpu.VMEM((_LP, 128), jnp.float32),
        pltpu.VMEM((_LP, 128), jnp.float32),
        pltpu.VMEM((_LP, 128), jnp.float32),
        pltpu.VMEM((_LP, 128), jnp.float32),
        pltpu.VMEM((_LP, 128), jnp.float32),
        pltpu.VMEM((_LP, 128), jnp.float32),
        pltpu.SemaphoreType.DMA,
        pltpu.SemaphoreType.DMA,
        pltpu.SemaphoreType.DMA,
        pltpu.SemaphoreType.DMA,
        pltpu.SemaphoreType.DMA,
        pltpu.SemaphoreType.DMA,
        pltpu.SemaphoreType.DMA,
        pltpu.SemaphoreType.DMA,
    ],
    compiler_params=pltpu.CompilerParams(use_tc_tiling_on_sc=False),
)
def _gather(tbla_hbm, tblb_hbm, idx_hbm, outa_hbm, outb_hbm, idx_v,
            a0, a1, a2, a3, b0, b1, b2, b3,
            g0, g1, g2, g3, s0, s1, s2, s3):
    bufa = (a0, a1, a2, a3)
    bufb = (b0, b1, b2, b3)
    gsem = (g0, g1, g2, g3)
    ssem = (s0, s1, s2, s3)
    wid = lax.axis_index("s") * _NC + lax.axis_index("c")
    base = wid * _ROWS_W
    pltpu.sync_copy(idx_hbm.at[pl.ds(wid, 1)], idx_v)

    def start_gathers(j, b):
        isl = idx_v.at[0, j]
        pltpu.async_copy(tbla_hbm.at[isl], bufa[b], gsem[b])
        pltpu.async_copy(tblb_hbm.at[isl], bufb[b], gsem[b])

    def wait_gathers(b):
        pltpu.make_async_copy(tbla_hbm.at[pl.ds(0, _LP)], bufa[b],
                              gsem[b]).wait()
        pltpu.make_async_copy(tblb_hbm.at[pl.ds(0, _LP)], bufb[b],
                              gsem[b]).wait()

    def start_stores(j, b):
        pltpu.async_copy(bufa[b], outa_hbm.at[base + j], ssem[b])
        pltpu.async_copy(bufb[b], outb_hbm.at[base + j], ssem[b])

    def wait_stores(b):
        pltpu.make_async_copy(bufa[b], outa_hbm.at[0], ssem[b]).wait()
        pltpu.make_async_copy(bufb[b], outb_hbm.at[0], ssem[b]).wait()

    for b in range(_NBUF):
        start_gathers(b, b)

    def group(g, carry):
        j0 = g * _NBUF
        for b in range(_NBUF):
            wait_gathers(b)
            start_stores(j0 + b, b)

            @pl.when(g + 1 < _GROUPS)
            def _():
                wait_stores(b)
                start_gathers(j0 + b + _NBUF, b)
        return carry

    lax.fori_loop(0, _GROUPS, group, 0)
    for b in range(_NBUF):
        wait_stores(b)


def kernel(indices, table):
    tbla = table[:, :128]
    tblb = jnp.pad(table[:, 128:], ((0, 0), (0, 128 - _DB)))
    idxp = jnp.pad(indices, ((0, 0), (0, _LP - _L))).reshape(
        _NW, _ROWS_W, _LP)
    outa, outb = _gather(tbla, tblb, idxp)
    return jnp.concatenate([outa[:, :_L, :], outb[:, :_L, :_DB]], axis=2)


# speedup vs baseline: 2.2483x; 2.2483x over previous
import functools

import jax
import jax.numpy as jnp
from jax import lax
from jax.experimental import pallas as pl
from jax.experimental.pallas import tpu as pltpu
from jax.experimental.pallas import tpu_sc as plsc

_NUM_Q = 100000
_D = 200
_B = 4096
_L = 50
_N = _B * _L
_DB = _D - 128

_info = plsc.get_sparse_core_info()
_NC = _info.num_cores
_NS = _info.num_subcores
_NW = _NC * _NS
_CH = 64
_NBUF = 4
_PER_W = _N // _NW
_STEPS = _PER_W // _CH
_GROUPS = _STEPS // _NBUF

_mesh = plsc.VectorSubcoreMesh(core_axis_name="c", subcore_axis_name="s")


@functools.partial(
    pl.kernel,
    out_type=(
        jax.ShapeDtypeStruct((_N, 128), jnp.float32),
        jax.ShapeDtypeStruct((_N, 128), jnp.float32),
    ),
    mesh=_mesh,
    scratch_types=[
        pltpu.VMEM((1, _STEPS, _CH), jnp.int32),
        pltpu.VMEM((_CH, 128), jnp.float32),
        pltpu.VMEM((_CH, 128), jnp.float32),
        pltpu.VMEM((_CH, 128), jnp.float32),
        pltpu.VMEM((_CH, 128), jnp.float32),
        pltpu.VMEM((_CH, 128), jnp.float32),
        pltpu.VMEM((_CH, 128), jnp.float32),
        pltpu.VMEM((_CH, 128), jnp.float32),
        pltpu.VMEM((_CH, 128), jnp.float32),
        pltpu.SemaphoreType.DMA,
        pltpu.SemaphoreType.DMA,
        pltpu.SemaphoreType.DMA,
        pltpu.SemaphoreType.DMA,
        pltpu.SemaphoreType.DMA,
        pltpu.SemaphoreType.DMA,
        pltpu.SemaphoreType.DMA,
        pltpu.SemaphoreType.DMA,
    ],
    compiler_params=pltpu.CompilerParams(use_tc_tiling_on_sc=False),
)
def _gather(tbla_hbm, tblb_hbm, idx_hbm, outa_hbm, outb_hbm, idx_v,
            a0, a1, a2, a3, b0, b1, b2, b3,
            g0, g1, g2, g3, s0, s1, s2, s3):
    bufa = (a0, a1, a2, a3)
    bufb = (b0, b1, b2, b3)
    gsem = (g0, g1, g2, g3)
    ssem = (s0, s1, s2, s3)
    wid = lax.axis_index("s") * _NC + lax.axis_index("c")
    base = wid * _PER_W
    pltpu.sync_copy(idx_hbm.at[pl.ds(wid, 1)], idx_v)

    def start_gathers(j, b):
        isl = idx_v.at[0, j]
        pltpu.async_copy(tbla_hbm.at[isl], bufa[b], gsem[b])
        pltpu.async_copy(tblb_hbm.at[isl], bufb[b], gsem[b])

    def wait_gathers(b):
        pltpu.make_async_copy(tbla_hbm.at[pl.ds(0, _CH)], bufa[b],
                              gsem[b]).wait()
        pltpu.make_async_copy(tblb_hbm.at[pl.ds(0, _CH)], bufb[b],
                              gsem[b]).wait()

    def start_stores(j, b):
        off = base + j * _CH
        pltpu.async_copy(bufa[b], outa_hbm.at[pl.ds(off, _CH)], ssem[b])
        pltpu.async_copy(bufb[b], outb_hbm.at[pl.ds(off, _CH)], ssem[b])

    def wait_stores(b):
        pltpu.make_async_copy(bufa[b], outa_hbm.at[pl.ds(0, _CH)],
                              ssem[b]).wait()
        pltpu.make_async_copy(bufb[b], outb_hbm.at[pl.ds(0, _CH)],
                              ssem[b]).wait()

    for b in range(_NBUF):
        start_gathers(b, b)

    def group(g, carry):
        j0 = g * _NBUF
        for b in range(_NBUF):
            wait_gathers(b)
            start_stores(j0 + b, b)

            @pl.when(g + 1 < _GROUPS)
            def _():
                wait_stores(b)
                start_gathers(j0 + b + _NBUF, b)
        return carry

    lax.fori_loop(0, _GROUPS, group, 0)
    for b in range(_NBUF):
        wait_stores(b)


def kernel(indices, table):
    tbla = table[:, :128]
    tblb = jnp.pad(table[:, 128:], ((0, 0), (0, 128 - _DB)))
    idx2 = indices.reshape(_NW, _STEPS, _CH)
    outa, outb = _gather(tbla, tblb, idx2)
    return jnp.concatenate([outa, outb[:, :_DB]], axis=1).reshape(
        _B, _L, _D)
